# baseline (speedup 1.0000x reference)
import jax
import jax.numpy as jnp
from jax.experimental import pallas as pl

N = 8192
D = 128
H = 8
DH = 16
KNN = 32
TAU_NEG = 0.6
BETA = 1.5
FFN = 192
EPS = 1e-5


def _ln(x, w, b):
    mu = jnp.mean(x, axis=-1, keepdims=True)
    var = jnp.mean((x - mu) ** 2, axis=-1, keepdims=True)
    return (x - mu) / jnp.sqrt(var + EPS) * w + b


def _identity_kernel(x_ref, o_ref):
    o_ref[...] = x_ref[...]


def kernel(features, coords, Wq, Wk, Wv, Wo, bo, ln1_w, ln1_b, W1, b1, W2, b2, ln2_w, ln2_b):
    n = features.shape[0]
    sq = jnp.sum(coords * coords, axis=1)
    d2 = sq[:, None] + sq[None, :] - 2.0 * (coords @ coords.T)
    dist = jnp.sqrt(jnp.maximum(d2, 0.0))
    _, idx = jax.lax.top_k(-dist, KNN + 1)
    knn = idx[:, 1:]
    neighbor_coords = coords[knn]
    sd = jnp.sqrt(jnp.sum((coords[:, None, :] - neighbor_coords) ** 2, axis=2))
    Q = (features @ Wq.T).reshape(n, H, DH)
    K = (features @ Wk.T).reshape(n, H, DH)
    V = (features @ Wv.T).reshape(n, H, DH)
    K_loc = jnp.transpose(K[knn], (0, 2, 1, 3))
    V_loc = jnp.transpose(V[knn], (0, 2, 1, 3))
    raw = jnp.einsum('nhd,nhkd->nhk', Q, K_loc) / (DH ** 0.5)
    scal = jnp.array([1.0 / 2.0 ** (h + 1) for h in range(H)], dtype=features.dtype)
    sc = sd[:, None, :] * scal[None, :, None]
    pos = jax.nn.softmax(raw - sc, axis=-1)
    neg = jax.nn.softmax((-raw - sc) / TAU_NEG, axis=-1)
    w = pos - BETA * neg
    att = jnp.einsum('nhk,nhkd->nhd', w, V_loc).reshape(n, D)
    x = att @ Wo.T + bo + features
    x = _ln(x, ln1_w, ln1_b)
    h = jax.nn.gelu(x @ W1.T + b1, approximate=False)
    ffn = h @ W2.T + b2
    out = _ln(ffn + x, ln2_w, ln2_b)
    return pl.pallas_call(
        _identity_kernel,
        out_shape=jax.ShapeDtypeStruct(out.shape, out.dtype),
    )(out)


# speedup vs baseline: 5.1608x; 5.1608x over previous
import functools

import jax
import jax.numpy as jnp
from jax import lax
from jax.experimental import pallas as pl
from jax.experimental.pallas import tpu as pltpu
from jax.experimental.pallas import tpu_sc as plsc

N = 8192
D = 128
H = 8
DH = 16
KNN_K = 32
TOPK = KNN_K + 1
TAU_NEG = 0.6
BETA = 1.5
FFN = 192
EPS = 1e-5

BQ = 128
CS = 64
NCH = N // CS
CAP = 8
NCAND = NCH * CAP
IDX_MASK = CS - 1
INT_BIG = 0x7FFFFFFF

BT = 128
BR = 1024


def _rne_bf16(x):
    u = lax.bitcast_convert_type(x, jnp.int32)
    t = u + 0x7FFF + jnp.bitwise_and(lax.shift_right_logical(u, 16), 1)
    return lax.bitcast_convert_type(jnp.bitwise_and(t, -65536), jnp.float32)


def _knn_body(xc_ref, xr_ref, yc_ref, yr_ref, idx_ref, cand_ref):
    q0 = pl.program_id(0) * BQ
    xq = xr_ref[:, pl.ds(q0, BQ)]
    yq = yr_ref[:, pl.ds(q0, BQ)]
    sqq = xq * xq + yq * yq
    xqb = _rne_bf16(xq)
    yqb = _rne_bf16(yq)

    def chunk_body(c, carry):
        r0 = pl.multiple_of(c * CS, CS)
        xc = xc_ref[pl.ds(r0, CS), :]
        yc = yc_ref[pl.ds(r0, CS), :]
        sqc = xc * xc + yc * yc
        dot = _rne_bf16(xc) * xqb + _rne_bf16(yc) * yqb
        d2 = sqc + sqq - 2.0 * dot
        d2 = jnp.maximum(d2, 0.0)
        pb = lax.bitcast_convert_type(d2, jnp.int32)
        riota = lax.broadcasted_iota(jnp.int32, (CS, BQ), 0)
        pd = jnp.bitwise_or(jnp.bitwise_and(pb, jnp.int32(~IDX_MASK)), riota)
        rows = []
        for _ in range(CAP):
            m = jnp.min(pd, axis=0, keepdims=True)
            rows.append(m)
            pd = jnp.where(pd == m, INT_BIG, pd)
        cand_ref[pl.ds(pl.multiple_of(c * CAP, CAP), CAP), :] = (
            jnp.concatenate(rows, axis=0))
        return carry

    lax.fori_loop(0, NCH, chunk_body, 0)

    oiota = lax.broadcasted_iota(jnp.int32, (TOPK, BQ), 0)

    def sel_body(t, carry):
        cand = cand_ref[...]
        val = jnp.bitwise_and(cand, jnp.int32(~IDX_MASK))
        m = jnp.min(val, axis=0, keepdims=True)
        riota = lax.broadcasted_iota(jnp.int32, (NCAND, BQ), 0)
        gid = (lax.shift_right_logical(riota, 3) * CS
               + jnp.bitwise_and(cand, jnp.int32(IDX_MASK)))
        hit = val == m
        gwin = jnp.min(jnp.where(hit, gid, INT_BIG), axis=0,
                       keepdims=True)
        cand_ref[...] = jnp.where(hit & (gid == gwin), INT_BIG, cand)
        idx_ref[...] = jnp.where(oiota == t,
                                 jnp.broadcast_to(gwin, (TOPK, BQ)),
                                 idx_ref[...])
        return carry

    lax.fori_loop(0, TOPK, sel_body, 0)


def _qkv_body(f_ref, wq_ref, wk_ref, wv_ref, q_ref, k_ref, v_ref):
    f = f_ref[...]
    q_ref[...] = jnp.dot(f, wq_ref[...], preferred_element_type=jnp.float32)
    k_ref[...] = jnp.dot(f, wk_ref[...], preferred_element_type=jnp.float32)
    v_ref[...] = jnp.dot(f, wv_ref[...], preferred_element_type=jnp.float32)


def _tail_body(f_ref, q_ref, kl_ref, vl_ref, sd2_ref,
               wo_ref, bo_ref, l1w_ref, l1b_ref,
               w1_ref, b1_ref, w2_ref, b2_ref, l2w_ref, l2b_ref, o_ref):
    K3 = kl_ref[...].reshape(BT, KNN_K, D)
    V3 = vl_ref[...].reshape(BT, KNN_K, D)
    Qb = q_ref[...]
    sdb = jnp.sqrt(sd2_ref[...])
    att_heads = []
    for h in range(H):
        Qh = Qb[:, h * DH:(h + 1) * DH]
        Kh = K3[:, :, h * DH:(h + 1) * DH]
        raw = jnp.sum(Kh * Qh[:, None, :], axis=2) / (DH ** 0.5)
        scb = sdb * (1.0 / 2.0 ** (h + 1))
        a = raw - scb
        am = jnp.max(a, axis=1, keepdims=True)
        ea = jnp.exp(a - am)
        pos = ea / jnp.sum(ea, axis=1, keepdims=True)
        b = (-raw - scb) / TAU_NEG
        bm = jnp.max(b, axis=1, keepdims=True)
        eb = jnp.exp(b - bm)
        neg = eb / jnp.sum(eb, axis=1, keepdims=True)
        w = pos - BETA * neg
        Vh = V3[:, :, h * DH:(h + 1) * DH]
        att_heads.append(jnp.sum(Vh * w[:, :, None], axis=1))
    att = jnp.concatenate(att_heads, axis=1)
    x = (jnp.dot(att, wo_ref[...], preferred_element_type=jnp.float32)
         + bo_ref[...] + f_ref[...])
    mu = jnp.mean(x, axis=1, keepdims=True)
    var = jnp.mean((x - mu) ** 2, axis=1, keepdims=True)
    xn = (x - mu) / jnp.sqrt(var + EPS) * l1w_ref[...] + l1b_ref[...]
    hfc = jnp.dot(xn, w1_ref[...], preferred_element_type=jnp.float32) + b1_ref[...]
    g = 0.5 * hfc * (1.0 + lax.erf(hfc * (2.0 ** -0.5)))
    f2 = jnp.dot(g, w2_ref[...], preferred_element_type=jnp.float32) + b2_ref[...]
    y = f2 + xn
    mu2 = jnp.mean(y, axis=1, keepdims=True)
    var2 = jnp.mean((y - mu2) ** 2, axis=1, keepdims=True)
    o_ref[...] = (y - mu2) / jnp.sqrt(var2 + EPS) * l2w_ref[...] + l2b_ref[...]


_GROWS = N * KNN_K
_NW = 32
_RPW = _GROWS // _NW
_GCH = 128
_NGCH = _RPW // _GCH


def _sc_gather_body(k_hbm, v_hbm, x_hbm, y_hbm, idx_hbm,
                    kl_hbm, vl_hbm, sd2_hbm,
                    idx_v, kbuf, vbuf, xbuf, ybuf, d2buf,
                    sem_k, sem_v):
    wid = lax.axis_index("s") * 2 + lax.axis_index("c")
    base = wid * _RPW
    pltpu.sync_copy(x_hbm, xbuf)
    pltpu.sync_copy(y_hbm, ybuf)

    def chunk(c, carry):
        off = base + c * _GCH
        pltpu.sync_copy(idx_hbm.at[pl.ds(off, _GCH)], idx_v)
        ck = pltpu.async_copy(k_hbm.at[idx_v], kbuf, sem_k)
        cv = pltpu.async_copy(v_hbm.at[idx_v], vbuf, sem_v)

        def lane(j, carry2):
            idxv = idx_v[pl.ds(j * 16, 16)]
            qidx = lax.shift_right_logical(
                off + j * 16 + lax.iota(jnp.int32, 16), 5)
            ih = lax.shift_right_logical(idxv, 7)
            il = jnp.bitwise_and(idxv, 127)
            qh = lax.shift_right_logical(qidx, 7)
            ql = jnp.bitwise_and(qidx, 127)
            xg = plsc.load_gather(xbuf, [ih, il])
            yg = plsc.load_gather(ybuf, [ih, il])
            xq = plsc.load_gather(xbuf, [qh, ql])
            yq = plsc.load_gather(ybuf, [qh, ql])
            dx = xq - xg
            dy = yq - yg
            d2buf[pl.ds(j * 16, 16)] = dx * dx + dy * dy
            return carry2

        lax.fori_loop(0, _GCH // 16, lane, 0)
        pltpu.sync_copy(d2buf, sd2_hbm.at[pl.ds(off, _GCH)])
        ck.wait()
        cv.wait()
        pltpu.sync_copy(kbuf, kl_hbm.at[pl.ds(off, _GCH)])
        pltpu.sync_copy(vbuf, vl_hbm.at[pl.ds(off, _GCH)])
        return carry

    lax.fori_loop(0, _NGCH, chunk, 0)


@functools.lru_cache(maxsize=1)
def _make_sc_gather():
    return pl.kernel(
        _sc_gather_body,
        out_type=(jax.ShapeDtypeStruct((_GROWS, D), jnp.float32),
                  jax.ShapeDtypeStruct((_GROWS, D), jnp.float32),
                  jax.ShapeDtypeStruct((_GROWS,), jnp.float32)),
        mesh=plsc.VectorSubcoreMesh(core_axis_name="c", subcore_axis_name="s"),
        compiler_params=pltpu.CompilerParams(needs_layout_passes=False),
        scratch_types=[
            pltpu.VMEM((_GCH,), jnp.int32),
            pltpu.VMEM((_GCH, D), jnp.float32),
            pltpu.VMEM((_GCH, D), jnp.float32),
            pltpu.VMEM((N // 128, 128), jnp.float32),
            pltpu.VMEM((N // 128, 128), jnp.float32),
            pltpu.VMEM((_GCH,), jnp.float32),
            pltpu.SemaphoreType.DMA,
            pltpu.SemaphoreType.DMA,
        ],
    )


def _sc_gather(k, v, x, y, idx):
    return _make_sc_gather()(k, v, x, y, idx)


def kernel(features, coords, Wq, Wk, Wv, Wo, bo, ln1_w, ln1_b,
           W1, b1, W2, b2, ln2_w, ln2_b):
    xc = coords[:, 0:1]
    yc = coords[:, 1:2]
    xr = coords[:, 0].reshape(1, N)
    yr = coords[:, 1].reshape(1, N)

    idx33 = pl.pallas_call(
        _knn_body,
        grid=(N // BQ,),
        in_specs=[
            pl.BlockSpec((N, 1), lambda i: (0, 0)),
            pl.BlockSpec((1, N), lambda i: (0, 0)),
            pl.BlockSpec((N, 1), lambda i: (0, 0)),
            pl.BlockSpec((1, N), lambda i: (0, 0)),
        ],
        out_specs=pl.BlockSpec((TOPK, BQ), lambda i: (0, i)),
        out_shape=jax.ShapeDtypeStruct((TOPK, N), jnp.int32),
        scratch_shapes=[pltpu.VMEM((NCAND, BQ), jnp.int32)],
    )(xc, xr, yc, yr)

    knn = idx33.T[:, 1:]

    q, k, v = pl.pallas_call(
        _qkv_body,
        grid=(N // BR,),
        in_specs=[
            pl.BlockSpec((BR, D), lambda i: (i, 0)),
            pl.BlockSpec((D, D), lambda i: (0, 0)),
            pl.BlockSpec((D, D), lambda i: (0, 0)),
            pl.BlockSpec((D, D), lambda i: (0, 0)),
        ],
        out_specs=[pl.BlockSpec((BR, D), lambda i: (i, 0))] * 3,
        out_shape=[jax.ShapeDtypeStruct((N, D), jnp.float32)] * 3,
    )(features, Wq.T, Wk.T, Wv.T)

    kl, vl, sd2 = _sc_gather(k, v,
                             coords[:, 0].reshape(N // 128, 128),
                             coords[:, 1].reshape(N // 128, 128),
                             knn.reshape(-1))
    sd2 = sd2.reshape(N, KNN_K)

    out = pl.pallas_call(
        _tail_body,
        grid=(N // BT,),
        in_specs=[
            pl.BlockSpec((BT, D), lambda i: (i, 0)),
            pl.BlockSpec((BT, D), lambda i: (i, 0)),
            pl.BlockSpec((BT * KNN_K, D), lambda i: (i, 0)),
            pl.BlockSpec((BT * KNN_K, D), lambda i: (i, 0)),
            pl.BlockSpec((BT, KNN_K), lambda i: (i, 0)),
            pl.BlockSpec((D, D), lambda i: (0, 0)),
            pl.BlockSpec((1, D), lambda i: (0, 0)),
            pl.BlockSpec((1, D), lambda i: (0, 0)),
            pl.BlockSpec((1, D), lambda i: (0, 0)),
            pl.BlockSpec((D, FFN), lambda i: (0, 0)),
            pl.BlockSpec((1, FFN), lambda i: (0, 0)),
            pl.BlockSpec((FFN, D), lambda i: (0, 0)),
            pl.BlockSpec((1, D), lambda i: (0, 0)),
            pl.BlockSpec((1, D), lambda i: (0, 0)),
            pl.BlockSpec((1, D), lambda i: (0, 0)),
        ],
        out_specs=pl.BlockSpec((BT, D), lambda i: (i, 0)),
        out_shape=jax.ShapeDtypeStruct((N, D), jnp.float32),
    )(features, q, kl, vl, sd2,
      Wo.T, bo.reshape(1, D), ln1_w.reshape(1, D), ln1_b.reshape(1, D),
      W1.T, b1.reshape(1, FFN), W2.T, b2.reshape(1, D),
      ln2_w.reshape(1, D), ln2_b.reshape(1, D))
    return out


# speedup vs baseline: 10.6337x; 2.0605x over previous
import functools

import jax
import jax.numpy as jnp
from jax import lax
from jax.experimental import pallas as pl
from jax.experimental.pallas import tpu as pltpu
from jax.experimental.pallas import tpu_sc as plsc

N = 8192
D = 128
H = 8
DH = 16
KNN_K = 32
TOPK = KNN_K + 1
TAU_NEG = 0.6
BETA = 1.5
FFN = 192
EPS = 1e-5

BQ = 128
CS = 64
NCH = N // CS
CAP = 8
NCAND = NCH * CAP
IDX_MASK = CS - 1
INT_BIG = 0x7FFFFFFF

BT = 128
BR = 1024


def _rne_bf16(x):
    u = lax.bitcast_convert_type(x, jnp.int32)
    t = u + 0x7FFF + jnp.bitwise_and(lax.shift_right_logical(u, 16), 1)
    return lax.bitcast_convert_type(jnp.bitwise_and(t, -65536), jnp.float32)


def _knn_body(xc_ref, xr_ref, yc_ref, yr_ref, idx_ref, cand_ref):
    q0 = pl.program_id(0) * BQ
    xq = xr_ref[:, pl.ds(q0, BQ)]
    yq = yr_ref[:, pl.ds(q0, BQ)]
    sqq = xq * xq + yq * yq
    xqb = _rne_bf16(xq)
    yqb = _rne_bf16(yq)

    def chunk_body(c, carry):
        r0 = pl.multiple_of(c * CS, CS)
        xc = xc_ref[pl.ds(r0, CS), :]
        yc = yc_ref[pl.ds(r0, CS), :]
        sqc = xc * xc + yc * yc
        dot = _rne_bf16(xc) * xqb + _rne_bf16(yc) * yqb
        d2 = sqc + sqq - 2.0 * dot
        d2 = jnp.maximum(d2, 0.0)
        pb = lax.bitcast_convert_type(d2, jnp.int32)
        riota = lax.broadcasted_iota(jnp.int32, (CS, BQ), 0)
        pd = jnp.bitwise_or(jnp.bitwise_and(pb, jnp.int32(~IDX_MASK)), riota)
        rows = []
        for _ in range(CAP):
            m = jnp.min(pd, axis=0, keepdims=True)
            rows.append(m)
            pd = jnp.where(pd == m, INT_BIG, pd)
        cand_ref[pl.ds(pl.multiple_of(c * CAP, CAP), CAP), :] = (
            jnp.concatenate(rows, axis=0))
        return carry

    lax.fori_loop(0, NCH, chunk_body, 0)

    oiota = lax.broadcasted_iota(jnp.int32, (TOPK, BQ), 0)

    def sel_body(t, carry):
        cand = cand_ref[...]
        val = jnp.bitwise_and(cand, jnp.int32(~IDX_MASK))
        m = jnp.min(val, axis=0, keepdims=True)
        riota = lax.broadcasted_iota(jnp.int32, (NCAND, BQ), 0)
        gid = (lax.shift_right_logical(riota, 3) * CS
               + jnp.bitwise_and(cand, jnp.int32(IDX_MASK)))
        hit = val == m
        gwin = jnp.min(jnp.where(hit, gid, INT_BIG), axis=0,
                       keepdims=True)
        cand_ref[...] = jnp.where(hit & (gid == gwin), INT_BIG, cand)
        idx_ref[...] = jnp.where(oiota == t,
                                 jnp.broadcast_to(gwin, (TOPK, BQ)),
                                 idx_ref[...])
        return carry

    lax.fori_loop(0, TOPK, sel_body, 0)


def _qkv_body(f_ref, wq_ref, wk_ref, wv_ref, q_ref, k_ref, v_ref):
    f = f_ref[...]
    q_ref[...] = jnp.dot(f, wq_ref[...], preferred_element_type=jnp.float32)
    k_ref[...] = jnp.dot(f, wk_ref[...], preferred_element_type=jnp.float32)
    v_ref[...] = jnp.dot(f, wv_ref[...], preferred_element_type=jnp.float32)


def _tail_body(f_ref, q_ref, kl_ref, vl_ref, sd2_ref,
               wo_ref, bo_ref, l1w_ref, l1b_ref,
               w1_ref, b1_ref, w2_ref, b2_ref, l2w_ref, l2b_ref, o_ref):
    K3 = kl_ref[...].reshape(BT, KNN_K, D)
    V3 = vl_ref[...].reshape(BT, KNN_K, D)
    Qb = q_ref[...]
    sdb = jnp.sqrt(sd2_ref[...])
    prod = (K3 * Qb[:, None, :]).reshape(BT * KNN_K, D)
    ci = lax.broadcasted_iota(jnp.int32, (D, D), 1)
    ri = lax.broadcasted_iota(jnp.int32, (D, D), 0)
    sel = (lax.shift_right_logical(ri, 4)
           == lax.shift_right_logical(ci, 4)).astype(jnp.float32)
    raw3 = (jnp.dot(prod, sel, preferred_element_type=jnp.float32)
            .reshape(BT, KNN_K, D)) * (1.0 / (DH ** 0.5))
    hc = lax.shift_right_logical(
        lax.broadcasted_iota(jnp.int32, (1, 1, D), 2), 4)
    scal = lax.bitcast_convert_type(
        lax.shift_left(126 - hc, 23), jnp.float32)
    sc3 = sdb[:, :, None] * scal
    a = raw3 - sc3
    am = jnp.max(a, axis=1, keepdims=True)
    ea = jnp.exp(a - am)
    pos = ea / jnp.sum(ea, axis=1, keepdims=True)
    b = (-raw3 - sc3) / TAU_NEG
    bm = jnp.max(b, axis=1, keepdims=True)
    eb = jnp.exp(b - bm)
    neg = eb / jnp.sum(eb, axis=1, keepdims=True)
    w3 = pos - BETA * neg
    att = jnp.sum(V3 * w3, axis=1)
    x = (jnp.dot(att, wo_ref[...], preferred_element_type=jnp.float32)
         + bo_ref[...] + f_ref[...])
    mu = jnp.mean(x, axis=1, keepdims=True)
    var = jnp.mean((x - mu) ** 2, axis=1, keepdims=True)
    xn = (x - mu) / jnp.sqrt(var + EPS) * l1w_ref[...] + l1b_ref[...]
    hfc = jnp.dot(xn, w1_ref[...], preferred_element_type=jnp.float32) + b1_ref[...]
    g = 0.5 * hfc * (1.0 + lax.erf(hfc * (2.0 ** -0.5)))
    f2 = jnp.dot(g, w2_ref[...], preferred_element_type=jnp.float32) + b2_ref[...]
    y = f2 + xn
    mu2 = jnp.mean(y, axis=1, keepdims=True)
    var2 = jnp.mean((y - mu2) ** 2, axis=1, keepdims=True)
    o_ref[...] = (y - mu2) / jnp.sqrt(var2 + EPS) * l2w_ref[...] + l2b_ref[...]


_GROWS = N * KNN_K
_NW = 32
_RPW = _GROWS // _NW
_GCH = 128
_NGCH = _RPW // _GCH


def _sc_gather_body(k_hbm, v_hbm, x_hbm, y_hbm, idx_hbm,
                    kl_hbm, vl_hbm, sd2_hbm,
                    idx_v, kbuf, vbuf, xbuf, ybuf, d2buf,
                    sem_k, sem_v):
    wid = lax.axis_index("s") * 2 + lax.axis_index("c")
    base = wid * _RPW
    pltpu.sync_copy(x_hbm, xbuf)
    pltpu.sync_copy(y_hbm, ybuf)

    def chunk(c, carry):
        off = base + c * _GCH
        pltpu.sync_copy(idx_hbm.at[pl.ds(off, _GCH)], idx_v)
        ck = pltpu.async_copy(k_hbm.at[idx_v], kbuf, sem_k)
        cv = pltpu.async_copy(v_hbm.at[idx_v], vbuf, sem_v)

        def lane(j, carry2):
            idxv = idx_v[pl.ds(j * 16, 16)]
            qidx = lax.shift_right_logical(
                off + j * 16 + lax.iota(jnp.int32, 16), 5)
            ih = lax.shift_right_logical(idxv, 7)
            il = jnp.bitwise_and(idxv, 127)
            qh = lax.shift_right_logical(qidx, 7)
            ql = jnp.bitwise_and(qidx, 127)
            xg = plsc.load_gather(xbuf, [ih, il])
            yg = plsc.load_gather(ybuf, [ih, il])
            xq = plsc.load_gather(xbuf, [qh, ql])
            yq = plsc.load_gather(ybuf, [qh, ql])
            dx = xq - xg
            dy = yq - yg
            d2buf[pl.ds(j * 16, 16)] = dx * dx + dy * dy
            return carry2

        lax.fori_loop(0, _GCH // 16, lane, 0)
        pltpu.sync_copy(d2buf, sd2_hbm.at[pl.ds(off, _GCH)])
        ck.wait()
        cv.wait()
        pltpu.sync_copy(kbuf, kl_hbm.at[pl.ds(off, _GCH)])
        pltpu.sync_copy(vbuf, vl_hbm.at[pl.ds(off, _GCH)])
        return carry

    lax.fori_loop(0, _NGCH, chunk, 0)


@functools.lru_cache(maxsize=1)
def _make_sc_gather():
    return pl.kernel(
        _sc_gather_body,
        out_type=(jax.ShapeDtypeStruct((_GROWS, D), jnp.float32),
                  jax.ShapeDtypeStruct((_GROWS, D), jnp.float32),
                  jax.ShapeDtypeStruct((_GROWS,), jnp.float32)),
        mesh=plsc.VectorSubcoreMesh(core_axis_name="c", subcore_axis_name="s"),
        compiler_params=pltpu.CompilerParams(needs_layout_passes=False),
        scratch_types=[
            pltpu.VMEM((_GCH,), jnp.int32),
            pltpu.VMEM((_GCH, D), jnp.float32),
            pltpu.VMEM((_GCH, D), jnp.float32),
            pltpu.VMEM((N // 128, 128), jnp.float32),
            pltpu.VMEM((N // 128, 128), jnp.float32),
            pltpu.VMEM((_GCH,), jnp.float32),
            pltpu.SemaphoreType.DMA,
            pltpu.SemaphoreType.DMA,
        ],
    )


def _sc_gather(k, v, x, y, idx):
    return _make_sc_gather()(k, v, x, y, idx)


def kernel(features, coords, Wq, Wk, Wv, Wo, bo, ln1_w, ln1_b,
           W1, b1, W2, b2, ln2_w, ln2_b):
    xc = coords[:, 0:1]
    yc = coords[:, 1:2]
    xr = coords[:, 0].reshape(1, N)
    yr = coords[:, 1].reshape(1, N)

    idx33 = pl.pallas_call(
        _knn_body,
        grid=(N // BQ,),
        in_specs=[
            pl.BlockSpec((N, 1), lambda i: (0, 0)),
            pl.BlockSpec((1, N), lambda i: (0, 0)),
            pl.BlockSpec((N, 1), lambda i: (0, 0)),
            pl.BlockSpec((1, N), lambda i: (0, 0)),
        ],
        out_specs=pl.BlockSpec((TOPK, BQ), lambda i: (0, i)),
        out_shape=jax.ShapeDtypeStruct((TOPK, N), jnp.int32),
        scratch_shapes=[pltpu.VMEM((NCAND, BQ), jnp.int32)],
    )(xc, xr, yc, yr)

    knn = idx33.T[:, 1:]

    q, k, v = pl.pallas_call(
        _qkv_body,
        grid=(N // BR,),
        in_specs=[
            pl.BlockSpec((BR, D), lambda i: (i, 0)),
            pl.BlockSpec((D, D), lambda i: (0, 0)),
            pl.BlockSpec((D, D), lambda i: (0, 0)),
            pl.BlockSpec((D, D), lambda i: (0, 0)),
        ],
        out_specs=[pl.BlockSpec((BR, D), lambda i: (i, 0))] * 3,
        out_shape=[jax.ShapeDtypeStruct((N, D), jnp.float32)] * 3,
    )(features, Wq.T, Wk.T, Wv.T)

    kl, vl, sd2 = _sc_gather(k, v,
                             coords[:, 0].reshape(N // 128, 128),
                             coords[:, 1].reshape(N // 128, 128),
                             knn.reshape(-1))
    sd2 = sd2.reshape(N, KNN_K)

    out = pl.pallas_call(
        _tail_body,
        grid=(N // BT,),
        in_specs=[
            pl.BlockSpec((BT, D), lambda i: (i, 0)),
            pl.BlockSpec((BT, D), lambda i: (i, 0)),
            pl.BlockSpec((BT * KNN_K, D), lambda i: (i, 0)),
            pl.BlockSpec((BT * KNN_K, D), lambda i: (i, 0)),
            pl.BlockSpec((BT, KNN_K), lambda i: (i, 0)),
            pl.BlockSpec((D, D), lambda i: (0, 0)),
            pl.BlockSpec((1, D), lambda i: (0, 0)),
            pl.BlockSpec((1, D), lambda i: (0, 0)),
            pl.BlockSpec((1, D), lambda i: (0, 0)),
            pl.BlockSpec((D, FFN), lambda i: (0, 0)),
            pl.BlockSpec((1, FFN), lambda i: (0, 0)),
            pl.BlockSpec((FFN, D), lambda i: (0, 0)),
            pl.BlockSpec((1, D), lambda i: (0, 0)),
            pl.BlockSpec((1, D), lambda i: (0, 0)),
            pl.BlockSpec((1, D), lambda i: (0, 0)),
        ],
        out_specs=pl.BlockSpec((BT, D), lambda i: (i, 0)),
        out_shape=jax.ShapeDtypeStruct((N, D), jnp.float32),
    )(features, q, kl, vl, sd2,
      Wo.T, bo.reshape(1, D), ln1_w.reshape(1, D), ln1_b.reshape(1, D),
      W1.T, b1.reshape(1, FFN), W2.T, b2.reshape(1, D),
      ln2_w.reshape(1, D), ln2_b.reshape(1, D))
    return out


# speedup vs baseline: 12.3050x; 1.1572x over previous
import functools

import jax
import jax.numpy as jnp
from jax import lax
from jax.experimental import pallas as pl
from jax.experimental.pallas import tpu as pltpu
from jax.experimental.pallas import tpu_sc as plsc

N = 8192
D = 128
H = 8
DH = 16
KNN_K = 32
TOPK = KNN_K + 1
TAU_NEG = 0.6
BETA = 1.5
FFN = 192
EPS = 1e-5

BQ = 128
CS = 64
NCH = N // CS
CAP = 8
NCAND = NCH * CAP
IDX_MASK = CS - 1
INT_BIG = 0x7FFFFFFF

BT = 128
BR = 1024


def _rne_bf16(x):
    u = lax.bitcast_convert_type(x, jnp.int32)
    t = u + 0x7FFF + jnp.bitwise_and(lax.shift_right_logical(u, 16), 1)
    return lax.bitcast_convert_type(jnp.bitwise_and(t, -65536), jnp.float32)


def _knn_body(xc_ref, xr_ref, yc_ref, yr_ref, idx_ref, cand_ref):
    q0 = pl.program_id(0) * BQ
    xq = xr_ref[:, pl.ds(q0, BQ)]
    yq = yr_ref[:, pl.ds(q0, BQ)]
    sqq = xq * xq + yq * yq
    xqb = _rne_bf16(xq)
    yqb = _rne_bf16(yq)

    def chunk_body(c, carry):
        r0 = pl.multiple_of(c * CS, CS)
        xc = xc_ref[pl.ds(r0, CS), :]
        yc = yc_ref[pl.ds(r0, CS), :]
        sqc = xc * xc + yc * yc
        dot = _rne_bf16(xc) * xqb + _rne_bf16(yc) * yqb
        d2 = sqc + sqq - 2.0 * dot
        d2 = jnp.maximum(d2, 0.0)
        pb = lax.bitcast_convert_type(d2, jnp.int32)
        riota = lax.broadcasted_iota(jnp.int32, (CS, BQ), 0)
        pd = jnp.bitwise_or(jnp.bitwise_and(pb, jnp.int32(~IDX_MASK)), riota)
        rows = []
        for _ in range(CAP):
            m = jnp.min(pd, axis=0, keepdims=True)
            rows.append(m)
            pd = jnp.where(pd == m, INT_BIG, pd)
        cand_ref[pl.ds(pl.multiple_of(c * CAP, CAP), CAP), :] = (
            jnp.concatenate(rows, axis=0))
        return carry

    lax.fori_loop(0, NCH, chunk_body, 0)

    oiota = lax.broadcasted_iota(jnp.int32, (TOPK, BQ), 0)

    cand0 = cand_ref[...]
    riota0 = lax.broadcasted_iota(jnp.int32, (NCAND, BQ), 0)
    gid0 = (lax.shift_right_logical(riota0, 3) * CS
            + jnp.bitwise_and(cand0, jnp.int32(IDX_MASK)))
    zero0 = jnp.bitwise_and(cand0, jnp.int32(~IDX_MASK)) == 0
    cand_ref[...] = jnp.where(zero0, gid0, cand0)

    def sel_body(t, carry):
        cand = cand_ref[...]
        m = jnp.min(cand, axis=0, keepdims=True)
        riota = lax.broadcasted_iota(jnp.int32, (NCAND, BQ), 0)
        hit = cand == m
        rwin = jnp.min(jnp.where(hit, riota, INT_BIG), axis=0,
                       keepdims=True)
        cand_ref[...] = jnp.where(riota == rwin, INT_BIG, cand)
        gnz = (lax.shift_right_logical(rwin, 3) * CS
               + jnp.bitwise_and(m, jnp.int32(IDX_MASK)))
        gwin = jnp.where(m < 8192, jnp.bitwise_and(m, jnp.int32(8191)), gnz)
        idx_ref[...] = jnp.where(oiota == t,
                                 jnp.broadcast_to(gwin, (TOPK, BQ)),
                                 idx_ref[...])
        return carry

    lax.fori_loop(0, TOPK, sel_body, 0)


def _qkv_body(f_ref, wq_ref, wk_ref, wv_ref, q_ref, k_ref, v_ref):
    f = f_ref[...]
    q_ref[...] = jnp.dot(f, wq_ref[...], preferred_element_type=jnp.float32)
    k_ref[...] = jnp.dot(f, wk_ref[...], preferred_element_type=jnp.float32)
    v_ref[...] = jnp.dot(f, wv_ref[...], preferred_element_type=jnp.float32)


def _tail_body(f_ref, q_ref, kl_ref, vl_ref, sd2_ref,
               wo_ref, bo_ref, l1w_ref, l1b_ref,
               w1_ref, b1_ref, w2_ref, b2_ref, l2w_ref, l2b_ref, o_ref):
    K3 = kl_ref[...].reshape(BT, KNN_K, D)
    V3 = vl_ref[...].reshape(BT, KNN_K, D)
    Qb = q_ref[...]
    sdb = jnp.sqrt(sd2_ref[...])
    prod = (K3 * Qb[:, None, :]).reshape(BT * KNN_K, D)
    ci = lax.broadcasted_iota(jnp.int32, (D, D), 1)
    ri = lax.broadcasted_iota(jnp.int32, (D, D), 0)
    sel = (lax.shift_right_logical(ri, 4)
           == lax.shift_right_logical(ci, 4)).astype(jnp.float32)
    raw3 = (jnp.dot(prod, sel, preferred_element_type=jnp.float32)
            .reshape(BT, KNN_K, D)) * (1.0 / (DH ** 0.5))
    hc = lax.shift_right_logical(
        lax.broadcasted_iota(jnp.int32, (1, 1, D), 2), 4)
    scal = lax.bitcast_convert_type(
        lax.shift_left(126 - hc, 23), jnp.float32)
    sc3 = sdb[:, :, None] * scal
    a = raw3 - sc3
    am = jnp.max(a, axis=1, keepdims=True)
    ea = jnp.exp(a - am)
    pos = ea / jnp.sum(ea, axis=1, keepdims=True)
    b = (-raw3 - sc3) / TAU_NEG
    bm = jnp.max(b, axis=1, keepdims=True)
    eb = jnp.exp(b - bm)
    neg = eb / jnp.sum(eb, axis=1, keepdims=True)
    w3 = pos - BETA * neg
    att = jnp.sum(V3 * w3, axis=1)
    x = (jnp.dot(att, wo_ref[...], preferred_element_type=jnp.float32)
         + bo_ref[...] + f_ref[...])
    mu = jnp.mean(x, axis=1, keepdims=True)
    var = jnp.mean((x - mu) ** 2, axis=1, keepdims=True)
    xn = (x - mu) / jnp.sqrt(var + EPS) * l1w_ref[...] + l1b_ref[...]
    hfc = jnp.dot(xn, w1_ref[...], preferred_element_type=jnp.float32) + b1_ref[...]
    g = 0.5 * hfc * (1.0 + lax.erf(hfc * (2.0 ** -0.5)))
    f2 = jnp.dot(g, w2_ref[...], preferred_element_type=jnp.float32) + b2_ref[...]
    y = f2 + xn
    mu2 = jnp.mean(y, axis=1, keepdims=True)
    var2 = jnp.mean((y - mu2) ** 2, axis=1, keepdims=True)
    o_ref[...] = (y - mu2) / jnp.sqrt(var2 + EPS) * l2w_ref[...] + l2b_ref[...]


_GROWS = N * KNN_K
_NW = 32
_RPW = _GROWS // _NW
_GCH = 128
_NGCH = _RPW // _GCH


def _sc_gather_body(k_hbm, v_hbm, x_hbm, y_hbm, idx_hbm,
                    kl_hbm, vl_hbm, sd2_hbm,
                    idx_v, kbuf, vbuf, xbuf, ybuf, d2buf,
                    sem_k, sem_v):
    wid = lax.axis_index("s") * 2 + lax.axis_index("c")
    base = wid * _RPW
    pltpu.sync_copy(x_hbm, xbuf)
    pltpu.sync_copy(y_hbm, ybuf)

    def chunk(c, carry):
        off = base + c * _GCH
        pltpu.sync_copy(idx_hbm.at[pl.ds(off, _GCH)], idx_v)
        ck = pltpu.async_copy(k_hbm.at[idx_v], kbuf, sem_k)
        cv = pltpu.async_copy(v_hbm.at[idx_v], vbuf, sem_v)

        def lane(j, carry2):
            idxv = idx_v[pl.ds(j * 16, 16)]
            qidx = lax.shift_right_logical(
                off + j * 16 + lax.iota(jnp.int32, 16), 5)
            ih = lax.shift_right_logical(idxv, 7)
            il = jnp.bitwise_and(idxv, 127)
            qh = lax.shift_right_logical(qidx, 7)
            ql = jnp.bitwise_and(qidx, 127)
            xg = plsc.load_gather(xbuf, [ih, il])
            yg = plsc.load_gather(ybuf, [ih, il])
            xq = plsc.load_gather(xbuf, [qh, ql])
            yq = plsc.load_gather(ybuf, [qh, ql])
            dx = xq - xg
            dy = yq - yg
            d2buf[pl.ds(j * 16, 16)] = dx * dx + dy * dy
            return carry2

        lax.fori_loop(0, _GCH // 16, lane, 0)
        pltpu.sync_copy(d2buf, sd2_hbm.at[pl.ds(off, _GCH)])
        ck.wait()
        cv.wait()
        pltpu.sync_copy(kbuf, kl_hbm.at[pl.ds(off, _GCH)])
        pltpu.sync_copy(vbuf, vl_hbm.at[pl.ds(off, _GCH)])
        return carry

    lax.fori_loop(0, _NGCH, chunk, 0)


@functools.lru_cache(maxsize=1)
def _make_sc_gather():
    return pl.kernel(
        _sc_gather_body,
        out_type=(jax.ShapeDtypeStruct((_GROWS, D), jnp.float32),
                  jax.ShapeDtypeStruct((_GROWS, D), jnp.float32),
                  jax.ShapeDtypeStruct((_GROWS,), jnp.float32)),
        mesh=plsc.VectorSubcoreMesh(core_axis_name="c", subcore_axis_name="s"),
        compiler_params=pltpu.CompilerParams(needs_layout_passes=False),
        scratch_types=[
            pltpu.VMEM((_GCH,), jnp.int32),
            pltpu.VMEM((_GCH, D), jnp.float32),
            pltpu.VMEM((_GCH, D), jnp.float32),
            pltpu.VMEM((N // 128, 128), jnp.float32),
            pltpu.VMEM((N // 128, 128), jnp.float32),
            pltpu.VMEM((_GCH,), jnp.float32),
            pltpu.SemaphoreType.DMA,
            pltpu.SemaphoreType.DMA,
        ],
    )


def _sc_gather(k, v, x, y, idx):
    return _make_sc_gather()(k, v, x, y, idx)


def kernel(features, coords, Wq, Wk, Wv, Wo, bo, ln1_w, ln1_b,
           W1, b1, W2, b2, ln2_w, ln2_b):
    xc = coords[:, 0:1]
    yc = coords[:, 1:2]
    xr = coords[:, 0].reshape(1, N)
    yr = coords[:, 1].reshape(1, N)

    idx33 = pl.pallas_call(
        _knn_body,
        grid=(N // BQ,),
        in_specs=[
            pl.BlockSpec((N, 1), lambda i: (0, 0)),
            pl.BlockSpec((1, N), lambda i: (0, 0)),
            pl.BlockSpec((N, 1), lambda i: (0, 0)),
            pl.BlockSpec((1, N), lambda i: (0, 0)),
        ],
        out_specs=pl.BlockSpec((TOPK, BQ), lambda i: (0, i)),
        out_shape=jax.ShapeDtypeStruct((TOPK, N), jnp.int32),
        scratch_shapes=[pltpu.VMEM((NCAND, BQ), jnp.int32)],
    )(xc, xr, yc, yr)

    knn = idx33.T[:, 1:]

    q, k, v = pl.pallas_call(
        _qkv_body,
        grid=(N // BR,),
        in_specs=[
            pl.BlockSpec((BR, D), lambda i: (i, 0)),
            pl.BlockSpec((D, D), lambda i: (0, 0)),
            pl.BlockSpec((D, D), lambda i: (0, 0)),
            pl.BlockSpec((D, D), lambda i: (0, 0)),
        ],
        out_specs=[pl.BlockSpec((BR, D), lambda i: (i, 0))] * 3,
        out_shape=[jax.ShapeDtypeStruct((N, D), jnp.float32)] * 3,
    )(features, Wq.T, Wk.T, Wv.T)

    kl, vl, sd2 = _sc_gather(k, v,
                             coords[:, 0].reshape(N // 128, 128),
                             coords[:, 1].reshape(N // 128, 128),
                             knn.reshape(-1))
    sd2 = sd2.reshape(N, KNN_K)

    out = pl.pallas_call(
        _tail_body,
        grid=(N // BT,),
        in_specs=[
            pl.BlockSpec((BT, D), lambda i: (i, 0)),
            pl.BlockSpec((BT, D), lambda i: (i, 0)),
            pl.BlockSpec((BT * KNN_K, D), lambda i: (i, 0)),
            pl.BlockSpec((BT * KNN_K, D), lambda i: (i, 0)),
            pl.BlockSpec((BT, KNN_K), lambda i: (i, 0)),
            pl.BlockSpec((D, D), lambda i: (0, 0)),
            pl.BlockSpec((1, D), lambda i: (0, 0)),
            pl.BlockSpec((1, D), lambda i: (0, 0)),
            pl.BlockSpec((1, D), lambda i: (0, 0)),
            pl.BlockSpec((D, FFN), lambda i: (0, 0)),
            pl.BlockSpec((1, FFN), lambda i: (0, 0)),
            pl.BlockSpec((FFN, D), lambda i: (0, 0)),
            pl.BlockSpec((1, D), lambda i: (0, 0)),
            pl.BlockSpec((1, D), lambda i: (0, 0)),
            pl.BlockSpec((1, D), lambda i: (0, 0)),
        ],
        out_specs=pl.BlockSpec((BT, D), lambda i: (i, 0)),
        out_shape=jax.ShapeDtypeStruct((N, D), jnp.float32),
    )(features, q, kl, vl, sd2,
      Wo.T, bo.reshape(1, D), ln1_w.reshape(1, D), ln1_b.reshape(1, D),
      W1.T, b1.reshape(1, FFN), W2.T, b2.reshape(1, D),
      ln2_w.reshape(1, D), ln2_b.reshape(1, D))
    return out
